# baseline (speedup 1.0000x reference)
import jax
import jax.numpy as jnp
from jax.experimental import pallas as pl
from jax.experimental.pallas import tpu as pltpu

N = 10000
E = 320000
D_IN = 128
H = 32
B = 100
K = 100
TOTAL_DIM = 3 * H + 1


def _gcn_conv(x, W, b, src, dst, n):
    h = x @ W
    loop = jnp.arange(n, dtype=src.dtype)
    s = jnp.concatenate([src, loop])
    d = jnp.concatenate([dst, loop])
    deg = jnp.zeros((n,), dtype=x.dtype).at[d].add(1.0)
    dinv = jnp.where(deg > 0, jax.lax.rsqrt(deg), 0.0)
    norm = (dinv[s] * dinv[d])[:, None]
    msg = h[s] * norm
    out = jnp.zeros((n, h.shape[1]), dtype=x.dtype).at[d].add(msg)
    return out + b


def _sort_pool(x, batch, num_graphs, k):
    n, d = x.shape
    perm = jnp.lexsort((-x[:, -1], batch))
    xs = x[perm]
    bs = batch[perm]
    counts = jnp.bincount(batch, length=num_graphs)
    starts = jnp.cumsum(counts) - counts
    pos = jnp.arange(n, dtype=bs.dtype) - starts[bs].astype(bs.dtype)
    mask = pos < k
    bi = jnp.where(mask, bs, num_graphs)
    pi = jnp.where(mask, pos, 0)
    full = jnp.zeros((num_graphs + 1, k, d), dtype=x.dtype).at[bi, pi].set(
        xs * mask[:, None].astype(x.dtype))
    return full[:num_graphs].reshape(num_graphs, k * d)


def _classifier_body(pk_ref, c1w_ref, c1b_ref, c2w_ref, c2b_ref,
                     f1w_ref, f1b_ref, f2w_ref, f2b_ref, out_ref):
    h1 = jnp.dot(pk_ref[...], c1w_ref[...],
                 preferred_element_type=jnp.float32,
                 precision=jax.lax.Precision.HIGHEST)
    h1 = jnp.maximum(h1 + c1b_ref[...][None, :], 0.0)
    h1 = h1.reshape(B, K // 2, 2, 16)
    h2 = jnp.maximum(h1[:, :, 0, :], h1[:, :, 1, :])
    cols = [h2[:, t:t + 46, :] for t in range(5)]
    w5 = jnp.concatenate(cols, axis=-1)
    h3 = jax.lax.dot_general(
        w5, c2w_ref[...], (((2,), (0,)), ((), ())),
        preferred_element_type=jnp.float32,
        precision=jax.lax.Precision.HIGHEST)
    h3 = jnp.maximum(h3 + c2b_ref[...][None, None, :], 0.0)
    h3 = jnp.transpose(h3, (0, 2, 1)).reshape(B, 32 * 46)
    h4 = jnp.dot(h3, f1w_ref[...], preferred_element_type=jnp.float32,
                 precision=jax.lax.Precision.HIGHEST) + f1b_ref[...][None, :]
    h4 = jnp.maximum(h4, 0.0)
    lg = jnp.dot(h4, f2w_ref[...], preferred_element_type=jnp.float32,
                 precision=jax.lax.Precision.HIGHEST) + f2b_ref[...][None, :]
    m = jnp.max(lg, axis=-1, keepdims=True)
    lse = m + jnp.log(jnp.sum(jnp.exp(lg - m), axis=-1, keepdims=True))
    out_ref[...] = lg - lse


def _classifier(pk, c1w, c1b, c2w, c2b, f1w, f1b, f2w, f2b):
    c1wr = c1w.reshape(16, TOTAL_DIM).T
    c2wr = jnp.transpose(c2w, (2, 1, 0)).reshape(80, 32)
    return pl.pallas_call(
        _classifier_body,
        out_shape=jax.ShapeDtypeStruct((B, 2), jnp.float32),
    )(pk, c1wr, c1b, c2wr, c2b, f1w, f1b, f2w, f2b)


def kernel(x, edge_index, batch, W1, b1, W2, b2, W3, b3, W4, b4,
           c1w, c1b, c2w, c2b, f1w, f1b, f2w, f2b):
    src = edge_index[0]
    dst = edge_index[1]
    x1 = jnp.tanh(_gcn_conv(x, W1, b1, src, dst, N))
    x2 = jnp.tanh(_gcn_conv(x1, W2, b2, src, dst, N))
    x3 = jnp.tanh(_gcn_conv(x2, W3, b3, src, dst, N))
    x4 = jnp.tanh(_gcn_conv(x3, W4, b4, src, dst, N))
    xc = jnp.concatenate([x1, x2, x3, x4], axis=1)
    p = _sort_pool(xc, batch, B, K)
    pk = p.reshape(B * K, TOTAL_DIM)
    return _classifier(pk, c1w, c1b, c2w, c2b, f1w, f1b, f2w, f2b)


# speedup vs baseline: 13.9351x; 13.9351x over previous
import functools

import jax
import jax.numpy as jnp
from jax import lax
from jax.experimental import pallas as pl
from jax.experimental.pallas import tpu as pltpu
from jax.experimental.pallas import tpu_sc as plsc

N = 10000
E = 320000
H = 32
B = 100
K = 100

NC = 2
NS = 16
NW = NC * NS
NPAD = N + 112
ECH = 128
NCH = 80
EPT = ECH * NCH
EP = EPT * NW
RPT = NPAD // NS
SR = 96
SLOTS = SR * 128
NB = 80
GB = 8
RB = NPAD // GB

_F32 = jnp.float32
_I32 = jnp.int32
_HI = jax.lax.Precision.HIGHEST


def _dot(a, b):
    return jnp.dot(a, b, preferred_element_type=_F32, precision=_HI)


def _sc_mesh():
    return plsc.VectorSubcoreMesh(core_axis_name="c", subcore_axis_name="s")


def _deg_pass(dstp, ones_rows, zrows):
    @functools.partial(
        pl.kernel,
        out_type=jax.ShapeDtypeStruct((NC, NPAD, 8), _F32),
        mesh=_sc_mesh(),
        compiler_params=pltpu.CompilerParams(use_tc_tiling_on_sc=False),
        scratch_types=[
            pltpu.VMEM((ECH,), _I32),
            pltpu.VMEM((ECH, 8), _F32),
            pltpu.VMEM_SHARED((NPAD, 8), _F32),
        ],
    )
    def k(dst_ref, ones_ref, z_ref, out_ref, didx, onesb, acc):
        cid = lax.axis_index("c")
        sid = lax.axis_index("s")
        base = (cid * NS + sid) * NCH
        pltpu.sync_copy(ones_ref, onesb)
        pltpu.sync_copy(z_ref, acc.at[pl.ds(sid * RPT, RPT)])
        plsc.subcore_barrier()

        def body(i, carry):
            pltpu.sync_copy(dst_ref.at[base + i], didx)
            pltpu.sync_copy(onesb, acc.at[didx], add=True)
            return carry

        lax.fori_loop(0, NCH, body, 0)
        plsc.subcore_barrier()
        pltpu.sync_copy(acc.at[pl.ds(sid * RPT, RPT)],
                        out_ref.at[cid, pl.ds(sid * RPT, RPT)])

    return k(dstp, ones_rows, zrows)


def _edge_pass(hs, srcp, dstp, zrows, F):
    @functools.partial(
        pl.kernel,
        out_type=jax.ShapeDtypeStruct((NC, NPAD, F), _F32),
        mesh=_sc_mesh(),
        compiler_params=pltpu.CompilerParams(use_tc_tiling_on_sc=False),
        scratch_types=[
            pltpu.VMEM((ECH,), _I32),
            pltpu.VMEM((ECH,), _I32),
            pltpu.VMEM((ECH, F), _F32),
            pltpu.VMEM_SHARED((NPAD, F), _F32),
        ],
    )
    def k(hs_ref, src_ref, dst_ref, z_ref, out_ref, sidx, didx, rows, acc):
        cid = lax.axis_index("c")
        sid = lax.axis_index("s")
        base = (cid * NS + sid) * NCH
        pltpu.sync_copy(z_ref, acc.at[pl.ds(sid * RPT, RPT)])
        plsc.subcore_barrier()

        def body(i, carry):
            g = base + i
            pltpu.sync_copy(src_ref.at[g], sidx)
            pltpu.sync_copy(dst_ref.at[g], didx)
            pltpu.sync_copy(hs_ref.at[sidx], rows)
            pltpu.sync_copy(rows, acc.at[didx], add=True)
            return carry

        lax.fori_loop(0, NCH, body, 0)
        plsc.subcore_barrier()
        pltpu.sync_copy(acc.at[pl.ds(sid * RPT, RPT)],
                        out_ref.at[cid, pl.ds(sid * RPT, RPT)])

    return k(hs, srcp, dstp, zrows)


def _pool_gather(ypad, gidx):
    @functools.partial(
        pl.kernel,
        out_type=jax.ShapeDtypeStruct((SLOTS, 16), _F32),
        mesh=_sc_mesh(),
        compiler_params=pltpu.CompilerParams(use_tc_tiling_on_sc=False),
        scratch_types=[
            pltpu.VMEM((128,), _I32),
            pltpu.VMEM((128, 16), _F32),
        ],
    )
    def k(y_ref, g_ref, out_ref, idxb, rows):
        cid = lax.axis_index("c")
        sid = lax.axis_index("s")
        tid = cid * NS + sid
        for j in range(SR // NW):
            r = tid * (SR // NW) + j
            pltpu.sync_copy(g_ref.at[r], idxb)
            pltpu.sync_copy(y_ref.at[idxb], rows)
            pltpu.sync_copy(rows, out_ref.at[pl.ds(r * 128, 128)])

    return k(ypad, gidx)


def _row_mask(i, f):
    thresh = N - i * RB
    return lax.broadcasted_iota(_I32, (RB, f), 0) < thresh


def _t1_body(x_ref, w_ref, dp_ref, dinv_ref, hs_ref):
    i = pl.program_id(0)
    deg8 = dp_ref[0] + dp_ref[1] + 1.0
    dinv8 = lax.rsqrt(deg8)
    dinv32 = jnp.concatenate([dinv8] * 4, axis=1)
    dinv_ref[...] = dinv32
    h = _dot(x_ref[...], w_ref[...])
    hs_ref[...] = jnp.where(_row_mask(i, H), dinv32 * h, 0.0)


def _tmid_body(p_ref, hsp_ref, dinv_ref, b_ref, w_ref, x_ref, hs_ref):
    i = pl.program_id(0)
    fo = w_ref.shape[1]
    s = p_ref[0] + p_ref[1] + hsp_ref[...]
    xl = jnp.tanh(dinv_ref[...] * s + b_ref[...])
    x_ref[...] = xl
    h = _dot(xl, w_ref[...])
    hs = dinv_ref[...][:, :fo] * h
    hs_ref[...] = jnp.where(_row_mask(i, fo), hs, 0.0)


def _t5_body(p_ref, hs4_ref, dinv_ref, b4_ref, x1_ref, x2_ref, x3_ref,
             c1a_ref, c1b2_ref, c1c_ref, c1d_ref, x4_ref, y_ref):
    i = pl.program_id(0)
    s4 = p_ref[0] + p_ref[1] + hs4_ref[...]
    x4 = jnp.tanh(dinv_ref[...][:, 0:8] * s4 + b4_ref[...])
    x4_ref[...] = x4
    x4_16 = jnp.concatenate([x4, x4], axis=1)
    y = (_dot(x1_ref[...], c1a_ref[...]) +
         _dot(x2_ref[...], c1b2_ref[...]) +
         _dot(x3_ref[...], c1c_ref[...]) +
         x4_16 * c1d_ref[...])
    y_ref[...] = jnp.where(_row_mask(i, 16), y, 0.0)


def _node_spec(f):
    return pl.BlockSpec((RB, f), lambda i: (i, 0))


def _part_spec(f):
    return pl.BlockSpec((2, RB, f), lambda i: (0, i, 0))


def _full_spec(shape):
    nd = len(shape)
    return pl.BlockSpec(shape, lambda i: (0,) * nd)


def _rank_body(v_ref, b_ref, gidx_ref, slots_ref):
    iota128 = lax.broadcasted_iota(_I32, (1, 128), 1)
    eye = (lax.broadcasted_iota(_I32, (128, 128), 0) ==
           lax.broadcasted_iota(_I32, (128, 128), 1)).astype(_F32)

    def tcol(row_f32):
        return lax.dot_general(eye, row_f32, (((1,), (1,)), ((), ())),
                               preferred_element_type=_F32, precision=_HI)

    def trow(col_f32):
        return lax.dot_general(col_f32, eye, (((0,), (0,)), ((), ())),
                               preferred_element_type=_F32, precision=_HI)

    counts = jnp.zeros((1, 128), _F32)
    for r in range(NB):
        b_col = tcol(b_ref[r:r + 1, :].astype(_F32))
        counts = counts + jnp.sum(
            (b_col == iota128.astype(_F32)).astype(_F32),
            axis=0, keepdims=True)
    tri = (lax.broadcasted_iota(_I32, (128, 128), 0) <
           lax.broadcasted_iota(_I32, (128, 128), 1)).astype(_F32)
    starts = _dot(counts, tri)
    ends = starts + counts

    for ci in range(NB):
        vi = tcol(v_ref[ci:ci + 1, :])
        bi = tcol(b_ref[ci:ci + 1, :].astype(_F32))
        ii = ci * 128 + lax.broadcasted_iota(_I32, (128, 1), 0)
        g_first = b_ref[ci, 0]
        g_last = b_ref[ci, 127]
        gf = g_first.astype(_F32)
        gl = g_last.astype(_F32)
        jlo = jnp.sum(jnp.where(iota128.astype(_F32) == gf, starts, 0.0))
        jhi = jnp.sum(jnp.where(iota128.astype(_F32) == gl, ends, 0.0))
        klo = jnp.floor(jlo / 128.0).astype(_I32)
        khi = jnp.ceil(jhi / 128.0).astype(_I32)

        def jbody(kk, acc, vi=vi, bi=bi, ii=ii):
            vj = v_ref[pl.ds(kk, 1), :]
            bj = b_ref[pl.ds(kk, 1), :].astype(_F32)
            ij = kk * 128 + iota128
            gt = (vj > vi) | ((vj == vi) & (ij < ii))
            hit = (gt & (bj == bi)).astype(_F32)
            return acc + jnp.sum(hit, axis=1, keepdims=True)

        rank = lax.fori_loop(klo, khi, jbody,
                             jnp.zeros((128, 1), _F32))
        slot = jnp.where((bi < float(B)) & (rank < float(K)),
                         bi * float(K) + rank, -1.0)
        slots_ref[ci:ci + 1, :] = trow(slot).astype(_I32)

    for rc in range(SR):
        s0 = rc * 128
        g0 = min(s0 // K, B - 1)
        g1 = min((s0 + 127) // K, B - 1)
        jlo = starts[0, g0]
        jhi = ends[0, g1]
        klo = jnp.floor(jlo / 128.0).astype(_I32)
        khi = jnp.ceil(jhi / 128.0).astype(_I32)
        svec = s0 + lax.broadcasted_iota(_I32, (128, 1), 0)

        def jbody2(kk, carry, svec=svec):
            gi_acc, w_acc = carry
            sj = slots_ref[pl.ds(kk, 1), :]
            ij = (kk * 128 + iota128).astype(_F32)
            eqm = (sj == svec).astype(_F32)
            gi_acc = gi_acc + jnp.sum(eqm * ij, axis=1, keepdims=True)
            w_acc = w_acc + jnp.sum(eqm, axis=1, keepdims=True)
            return gi_acc, w_acc

        gi, w = lax.fori_loop(klo, khi, jbody2,
                              (jnp.zeros((128, 1), _F32),
                               jnp.zeros((128, 1), _F32)))
        fill = (N + (svec % 32)).astype(_F32)
        out = jnp.where(w > 0, gi, fill)
        gidx_ref[rc:rc + 1, :] = trow(out).astype(_I32)


def _classifier_body(pool_ref, c1b_ref, c2w_ref, c2b_ref,
                     f1w_ref, f1b_ref, f2w_ref, f2b_ref, out_ref):
    h1 = pool_ref[...][:B * K] + c1b_ref[...]
    h1 = jnp.maximum(h1, 0.0).reshape(B, K // 2, 2, 16)
    h2 = jnp.maximum(h1[:, :, 0, :], h1[:, :, 1, :])
    w5 = jnp.concatenate([h2[:, t:t + 46, :] for t in range(5)], axis=-1)
    h3 = lax.dot_general(w5, c2w_ref[...], (((2,), (0,)), ((), ())),
                         preferred_element_type=_F32, precision=_HI)
    h3 = jnp.maximum(h3 + c2b_ref[...][None, None, :], 0.0)
    h3 = jnp.transpose(h3, (0, 2, 1)).reshape(B, 32 * 46)
    h4 = jnp.maximum(_dot(h3, f1w_ref[...]) + f1b_ref[...][None, :], 0.0)
    lg = _dot(h4, f2w_ref[...]) + f2b_ref[...][None, :]
    m = jnp.max(lg, axis=-1, keepdims=True)
    lse = m + jnp.log(jnp.sum(jnp.exp(lg - m), axis=-1, keepdims=True))
    out_ref[...] = lg - lse


def kernel(x, edge_index, batch, W1, b1, W2, b2, W3, b3, W4, b4,
           c1w, c1b, c2w, c2b, f1w, f1b, f2w, f2b):
    src = edge_index[0]
    dst = edge_index[1]

    fill = (N + (jnp.arange(EP - E, dtype=_I32) % 32))
    srcp = jnp.concatenate([src, fill]).reshape(EP // ECH, ECH)
    dstp = jnp.concatenate([dst, fill]).reshape(EP // ECH, ECH)
    batchp = jnp.concatenate(
        [batch, jnp.full((NB * 128 - N,), B, _I32)]).reshape(NB, 128)
    xpad = jnp.concatenate([x, jnp.zeros((NPAD - N, x.shape[1]), _F32)])
    z32 = jnp.zeros((RPT, H), _F32)
    z8 = jnp.zeros((RPT, 8), _F32)
    ones8 = jnp.ones((ECH, 8), _F32)
    W4rep = jnp.repeat(W4, 8, axis=1)
    b1r, b2r, b3r = b1.reshape(1, H), b2.reshape(1, H), b3.reshape(1, H)
    b4r = jnp.repeat(b4.reshape(1, 1), 8, axis=1)
    c1wr = c1w.reshape(16, 3 * H + 1).T
    c1a, c1b2, c1c = c1wr[0:H], c1wr[H:2 * H], c1wr[2 * H:3 * H]
    c1d = c1wr[3 * H:3 * H + 1]
    c2wr = jnp.transpose(c2w, (2, 1, 0)).reshape(80, 32)

    dp = _deg_pass(dstp, ones8, z8)

    dinv, hs1 = pl.pallas_call(
        _t1_body,
        grid=(GB,),
        in_specs=[_node_spec(128), _full_spec((128, H)), _part_spec(8)],
        out_specs=(_node_spec(H), _node_spec(H)),
        out_shape=(jax.ShapeDtypeStruct((NPAD, H), _F32),
                   jax.ShapeDtypeStruct((NPAD, H), _F32)),
    )(xpad, W1, dp)
    p1 = _edge_pass(hs1, srcp, dstp, z32, H)

    def mid(p, hsp, b, w):
        fo = w.shape[1]
        return pl.pallas_call(
            _tmid_body,
            grid=(GB,),
            in_specs=[_part_spec(H), _node_spec(H), _node_spec(H),
                      _full_spec((1, H)), _full_spec((H, fo))],
            out_specs=(_node_spec(H), _node_spec(fo)),
            out_shape=(jax.ShapeDtypeStruct((NPAD, H), _F32),
                       jax.ShapeDtypeStruct((NPAD, fo), _F32)),
        )(p, hsp, dinv, b, w)

    x1, hs2 = mid(p1, hs1, b1r, W2)
    p2 = _edge_pass(hs2, srcp, dstp, z32, H)
    x2, hs3 = mid(p2, hs2, b2r, W3)
    p3 = _edge_pass(hs3, srcp, dstp, z32, H)
    x3, hs4 = mid(p3, hs3, b3r, W4rep)
    p4 = _edge_pass(hs4, srcp, dstp, z8, 8)

    x4, ypad = pl.pallas_call(
        _t5_body,
        grid=(GB,),
        in_specs=[_part_spec(8), _node_spec(8), _node_spec(H),
                  _full_spec((1, 8)), _node_spec(H), _node_spec(H),
                  _node_spec(H), _full_spec((H, 16)), _full_spec((H, 16)),
                  _full_spec((H, 16)), _full_spec((1, 16))],
        out_specs=(_node_spec(8), _node_spec(16)),
        out_shape=(jax.ShapeDtypeStruct((NPAD, 8), _F32),
                   jax.ShapeDtypeStruct((NPAD, 16), _F32)),
    )(p4, hs4, dinv, b4r, x1, x2, x3, c1a, c1b2, c1c, c1d)

    vals2d = jnp.concatenate(
        [x4[:N, 0], jnp.zeros((NB * 128 - N,), _F32)]).reshape(NB, 128)
    gidx = pl.pallas_call(
        _rank_body,
        out_shape=jax.ShapeDtypeStruct((SR, 128), _I32),
        scratch_shapes=[pltpu.VMEM((NB, 128), _I32)],
    )(vals2d, batchp)

    pool16 = _pool_gather(ypad, gidx)

    return pl.pallas_call(
        _classifier_body,
        out_shape=jax.ShapeDtypeStruct((B, 2), _F32),
    )(pool16, c1b.reshape(1, 16), c2wr, c2b, f1w, f1b, f2w, f2b)


# speedup vs baseline: 26.4178x; 1.8958x over previous
import functools

import jax
import jax.numpy as jnp
from jax import lax
from jax.experimental import pallas as pl
from jax.experimental.pallas import tpu as pltpu
from jax.experimental.pallas import tpu_sc as plsc

N = 10000
E = 320000
H = 32
B = 100
K = 100

NC = 2
NS = 16
NW = NC * NS
NPAD = N + 112
ECH = 128
NCH = 80
EPT = ECH * NCH
EP = EPT * NW
RPT = NPAD // NS
SR = 96
SLOTS = SR * 128
NB = 80
GB = 8
RB = NPAD // GB

_F32 = jnp.float32
_I32 = jnp.int32
_HI = jax.lax.Precision.HIGHEST


def _dot(a, b):
    return jnp.dot(a, b, preferred_element_type=_F32, precision=_HI)


def _sc_mesh():
    return plsc.VectorSubcoreMesh(core_axis_name="c", subcore_axis_name="s")


def _deg_pass(dstp, ones_rows, zrows):
    @functools.partial(
        pl.kernel,
        out_type=jax.ShapeDtypeStruct((NC, NPAD, 8), _F32),
        mesh=_sc_mesh(),
        compiler_params=pltpu.CompilerParams(use_tc_tiling_on_sc=False),
        scratch_types=[
            pltpu.VMEM((ECH,), _I32),
            pltpu.VMEM((ECH, 8), _F32),
            pltpu.VMEM_SHARED((NPAD, 8), _F32),
        ],
    )
    def k(dst_ref, ones_ref, z_ref, out_ref, didx, onesb, acc):
        cid = lax.axis_index("c")
        sid = lax.axis_index("s")
        base = (cid * NS + sid) * NCH
        pltpu.sync_copy(ones_ref, onesb)
        pltpu.sync_copy(z_ref, acc.at[pl.ds(sid * RPT, RPT)])
        plsc.subcore_barrier()

        def body(i, carry):
            pltpu.sync_copy(dst_ref.at[base + i], didx)
            pltpu.sync_copy(onesb, acc.at[didx], add=True)
            return carry

        lax.fori_loop(0, NCH, body, 0)
        plsc.subcore_barrier()
        pltpu.sync_copy(acc.at[pl.ds(sid * RPT, RPT)],
                        out_ref.at[cid, pl.ds(sid * RPT, RPT)])

    return k(dstp, ones_rows, zrows)


_D = 8


def _edge_pass(hs, srcp, dstp, zrows, F):
    G = NCH // _D

    @functools.partial(
        pl.kernel,
        out_type=jax.ShapeDtypeStruct((NC, NPAD, F), _F32),
        mesh=_sc_mesh(),
        compiler_params=pltpu.CompilerParams(use_tc_tiling_on_sc=False),
        scratch_types=[
            pltpu.VMEM((NCH, ECH), _I32),
            pltpu.VMEM((NCH, ECH), _I32),
            pltpu.VMEM((_D, ECH, F), _F32),
            pltpu.VMEM_SHARED((NPAD, F), _F32),
        ] + [pltpu.SemaphoreType.DMA] * (2 * _D),
    )
    def k(hs_ref, src_ref, dst_ref, z_ref, out_ref, sidx, didx, rows, acc,
          *sems):
        gsem = sems[:_D]
        ssem = sems[_D:]
        cid = lax.axis_index("c")
        sid = lax.axis_index("s")
        base = (cid * NS + sid) * NCH
        pltpu.sync_copy(z_ref, acc.at[pl.ds(sid * RPT, RPT)])
        pltpu.sync_copy(src_ref.at[pl.ds(base, NCH)], sidx)
        pltpu.sync_copy(dst_ref.at[pl.ds(base, NCH)], didx)
        plsc.subcore_barrier()

        def gather(i, d):
            return pltpu.make_async_copy(hs_ref.at[sidx.at[i]], rows.at[d],
                                         gsem[d])

        def scat(i, d):
            return pltpu.make_async_copy(rows.at[d], acc.at[didx.at[i]],
                                         ssem[d])

        for d in range(_D):
            gather(d, d).start()

        def body(g, carry):
            i0 = g * _D
            for d in range(_D):
                gather(i0 + d, d).wait()
                scat(i0 + d, d).start(add=True)
            for d in range(_D):
                scat(i0 + d, d).wait()
                @pl.when(g < G - 1)
                def _():
                    gather(i0 + _D + d, d).start()
            return carry

        lax.fori_loop(0, G, body, 0)
        plsc.subcore_barrier()
        pltpu.sync_copy(acc.at[pl.ds(sid * RPT, RPT)],
                        out_ref.at[cid, pl.ds(sid * RPT, RPT)])

    return k(hs, srcp, dstp, zrows)


def _pool_gather(ypad, gidx):
    @functools.partial(
        pl.kernel,
        out_type=jax.ShapeDtypeStruct((SLOTS, 16), _F32),
        mesh=_sc_mesh(),
        compiler_params=pltpu.CompilerParams(use_tc_tiling_on_sc=False),
        scratch_types=[
            pltpu.VMEM((128,), _I32),
            pltpu.VMEM((128, 16), _F32),
        ],
    )
    def k(y_ref, g_ref, out_ref, idxb, rows):
        cid = lax.axis_index("c")
        sid = lax.axis_index("s")
        tid = cid * NS + sid
        for j in range(SR // NW):
            r = tid * (SR // NW) + j
            pltpu.sync_copy(g_ref.at[r], idxb)
            pltpu.sync_copy(y_ref.at[idxb], rows)
            pltpu.sync_copy(rows, out_ref.at[pl.ds(r * 128, 128)])

    return k(ypad, gidx)


def _row_mask(i, f):
    thresh = N - i * RB
    return lax.broadcasted_iota(_I32, (RB, f), 0) < thresh


def _t1_body(x_ref, w_ref, dp_ref, dinv_ref, hs_ref):
    i = pl.program_id(0)
    deg8 = dp_ref[0] + dp_ref[1] + 1.0
    dinv8 = lax.rsqrt(deg8)
    dinv32 = jnp.concatenate([dinv8] * 4, axis=1)
    dinv_ref[...] = dinv32
    h = _dot(x_ref[...], w_ref[...])
    hs_ref[...] = jnp.where(_row_mask(i, H), dinv32 * h, 0.0)


def _tmid_body(p_ref, hsp_ref, dinv_ref, b_ref, w_ref, x_ref, hs_ref):
    i = pl.program_id(0)
    fo = w_ref.shape[1]
    s = p_ref[0] + p_ref[1] + hsp_ref[...]
    xl = jnp.tanh(dinv_ref[...] * s + b_ref[...])
    x_ref[...] = xl
    h = _dot(xl, w_ref[...])
    hs = dinv_ref[...][:, :fo] * h
    hs_ref[...] = jnp.where(_row_mask(i, fo), hs, 0.0)


def _t5_body(p_ref, hs4_ref, dinv_ref, b4_ref, x1_ref, x2_ref, x3_ref,
             c1a_ref, c1b2_ref, c1c_ref, c1d_ref, x4_ref, y_ref):
    i = pl.program_id(0)
    s4 = p_ref[0] + p_ref[1] + hs4_ref[...]
    x4 = jnp.tanh(dinv_ref[...][:, 0:8] * s4 + b4_ref[...])
    x4_ref[...] = x4
    x4_16 = jnp.concatenate([x4, x4], axis=1)
    y = (_dot(x1_ref[...], c1a_ref[...]) +
         _dot(x2_ref[...], c1b2_ref[...]) +
         _dot(x3_ref[...], c1c_ref[...]) +
         x4_16 * c1d_ref[...])
    y_ref[...] = jnp.where(_row_mask(i, 16), y, 0.0)


def _node_spec(f):
    return pl.BlockSpec((RB, f), lambda i: (i, 0))


def _part_spec(f):
    return pl.BlockSpec((2, RB, f), lambda i: (0, i, 0))


def _full_spec(shape):
    nd = len(shape)
    return pl.BlockSpec(shape, lambda i: (0,) * nd)


def _rank_body(v_ref, b_ref, gidx_ref, slots_ref):
    iota128 = lax.broadcasted_iota(_I32, (1, 128), 1)
    eye = (lax.broadcasted_iota(_I32, (128, 128), 0) ==
           lax.broadcasted_iota(_I32, (128, 128), 1)).astype(_F32)

    def tcol(row_f32):
        return lax.dot_general(eye, row_f32, (((1,), (1,)), ((), ())),
                               preferred_element_type=_F32, precision=_HI)

    def trow(col_f32):
        return lax.dot_general(col_f32, eye, (((0,), (0,)), ((), ())),
                               preferred_element_type=_F32, precision=_HI)

    counts = jnp.zeros((1, 128), _F32)
    for r in range(NB):
        b_col = tcol(b_ref[r:r + 1, :].astype(_F32))
        counts = counts + jnp.sum(
            (b_col == iota128.astype(_F32)).astype(_F32),
            axis=0, keepdims=True)
    tri = (lax.broadcasted_iota(_I32, (128, 128), 0) <
           lax.broadcasted_iota(_I32, (128, 128), 1)).astype(_F32)
    starts = _dot(counts, tri)
    ends = starts + counts

    for ci in range(NB):
        vi = tcol(v_ref[ci:ci + 1, :])
        bi = tcol(b_ref[ci:ci + 1, :].astype(_F32))
        ii = ci * 128 + lax.broadcasted_iota(_I32, (128, 1), 0)
        g_first = b_ref[ci, 0]
        g_last = b_ref[ci, 127]
        gf = g_first.astype(_F32)
        gl = g_last.astype(_F32)
        jlo = jnp.sum(jnp.where(iota128.astype(_F32) == gf, starts, 0.0))
        jhi = jnp.sum(jnp.where(iota128.astype(_F32) == gl, ends, 0.0))
        klo = jnp.floor(jlo / 128.0).astype(_I32)
        khi = jnp.ceil(jhi / 128.0).astype(_I32)

        def jbody(kk, acc, vi=vi, bi=bi, ii=ii):
            vj = v_ref[pl.ds(kk, 1), :]
            bj = b_ref[pl.ds(kk, 1), :].astype(_F32)
            ij = kk * 128 + iota128
            gt = (vj > vi) | ((vj == vi) & (ij < ii))
            hit = (gt & (bj == bi)).astype(_F32)
            return acc + jnp.sum(hit, axis=1, keepdims=True)

        rank = lax.fori_loop(klo, khi, jbody,
                             jnp.zeros((128, 1), _F32))
        slot = jnp.where((bi < float(B)) & (rank < float(K)),
                         bi * float(K) + rank, -1.0)
        slots_ref[ci:ci + 1, :] = trow(slot).astype(_I32)

    for rc in range(SR):
        s0 = rc * 128
        g0 = min(s0 // K, B - 1)
        g1 = min((s0 + 127) // K, B - 1)
        jlo = starts[0, g0]
        jhi = ends[0, g1]
        klo = jnp.floor(jlo / 128.0).astype(_I32)
        khi = jnp.ceil(jhi / 128.0).astype(_I32)
        svec = s0 + lax.broadcasted_iota(_I32, (128, 1), 0)

        def jbody2(kk, carry, svec=svec):
            gi_acc, w_acc = carry
            sj = slots_ref[pl.ds(kk, 1), :]
            ij = (kk * 128 + iota128).astype(_F32)
            eqm = (sj == svec).astype(_F32)
            gi_acc = gi_acc + jnp.sum(eqm * ij, axis=1, keepdims=True)
            w_acc = w_acc + jnp.sum(eqm, axis=1, keepdims=True)
            return gi_acc, w_acc

        gi, w = lax.fori_loop(klo, khi, jbody2,
                              (jnp.zeros((128, 1), _F32),
                               jnp.zeros((128, 1), _F32)))
        fill = (N + (svec % 32)).astype(_F32)
        out = jnp.where(w > 0, gi, fill)
        gidx_ref[rc:rc + 1, :] = trow(out).astype(_I32)


def _classifier_body(pool_ref, c1b_ref, c2w_ref, c2b_ref,
                     f1w_ref, f1b_ref, f2w_ref, f2b_ref, out_ref):
    h1 = pool_ref[...][:B * K] + c1b_ref[...]
    h1 = jnp.maximum(h1, 0.0).reshape(B, K // 2, 2, 16)
    h2 = jnp.maximum(h1[:, :, 0, :], h1[:, :, 1, :])
    w5 = jnp.concatenate([h2[:, t:t + 46, :] for t in range(5)], axis=-1)
    h3 = lax.dot_general(w5, c2w_ref[...], (((2,), (0,)), ((), ())),
                         preferred_element_type=_F32, precision=_HI)
    h3 = jnp.maximum(h3 + c2b_ref[...][None, None, :], 0.0)
    h3 = jnp.transpose(h3, (0, 2, 1)).reshape(B, 32 * 46)
    h4 = jnp.maximum(_dot(h3, f1w_ref[...]) + f1b_ref[...][None, :], 0.0)
    lg = _dot(h4, f2w_ref[...]) + f2b_ref[...][None, :]
    m = jnp.max(lg, axis=-1, keepdims=True)
    lse = m + jnp.log(jnp.sum(jnp.exp(lg - m), axis=-1, keepdims=True))
    out_ref[...] = lg - lse


def kernel(x, edge_index, batch, W1, b1, W2, b2, W3, b3, W4, b4,
           c1w, c1b, c2w, c2b, f1w, f1b, f2w, f2b):
    src = edge_index[0]
    dst = edge_index[1]

    fill = (N + (jnp.arange(EP - E, dtype=_I32) % 32))
    srcp = jnp.concatenate([src, fill]).reshape(EP // ECH, ECH)
    dstp = jnp.concatenate([dst, fill]).reshape(EP // ECH, ECH)
    batchp = jnp.concatenate(
        [batch, jnp.full((NB * 128 - N,), B, _I32)]).reshape(NB, 128)
    xpad = jnp.concatenate([x, jnp.zeros((NPAD - N, x.shape[1]), _F32)])
    z32 = jnp.zeros((RPT, H), _F32)
    z8 = jnp.zeros((RPT, 8), _F32)
    ones8 = jnp.ones((ECH, 8), _F32)
    W4rep = jnp.repeat(W4, 8, axis=1)
    b1r, b2r, b3r = b1.reshape(1, H), b2.reshape(1, H), b3.reshape(1, H)
    b4r = jnp.repeat(b4.reshape(1, 1), 8, axis=1)
    c1wr = c1w.reshape(16, 3 * H + 1).T
    c1a, c1b2, c1c = c1wr[0:H], c1wr[H:2 * H], c1wr[2 * H:3 * H]
    c1d = c1wr[3 * H:3 * H + 1]
    c2wr = jnp.transpose(c2w, (2, 1, 0)).reshape(80, 32)

    dp = _deg_pass(dstp, ones8, z8)

    dinv, hs1 = pl.pallas_call(
        _t1_body,
        grid=(GB,),
        in_specs=[_node_spec(128), _full_spec((128, H)), _part_spec(8)],
        out_specs=(_node_spec(H), _node_spec(H)),
        out_shape=(jax.ShapeDtypeStruct((NPAD, H), _F32),
                   jax.ShapeDtypeStruct((NPAD, H), _F32)),
    )(xpad, W1, dp)
    p1 = _edge_pass(hs1, srcp, dstp, z32, H)

    def mid(p, hsp, b, w):
        fo = w.shape[1]
        return pl.pallas_call(
            _tmid_body,
            grid=(GB,),
            in_specs=[_part_spec(H), _node_spec(H), _node_spec(H),
                      _full_spec((1, H)), _full_spec((H, fo))],
            out_specs=(_node_spec(H), _node_spec(fo)),
            out_shape=(jax.ShapeDtypeStruct((NPAD, H), _F32),
                       jax.ShapeDtypeStruct((NPAD, fo), _F32)),
        )(p, hsp, dinv, b, w)

    x1, hs2 = mid(p1, hs1, b1r, W2)
    p2 = _edge_pass(hs2, srcp, dstp, z32, H)
    x2, hs3 = mid(p2, hs2, b2r, W3)
    p3 = _edge_pass(hs3, srcp, dstp, z32, H)
    x3, hs4 = mid(p3, hs3, b3r, W4rep)
    p4 = _edge_pass(hs4, srcp, dstp, z8, 8)

    x4, ypad = pl.pallas_call(
        _t5_body,
        grid=(GB,),
        in_specs=[_part_spec(8), _node_spec(8), _node_spec(H),
                  _full_spec((1, 8)), _node_spec(H), _node_spec(H),
                  _node_spec(H), _full_spec((H, 16)), _full_spec((H, 16)),
                  _full_spec((H, 16)), _full_spec((1, 16))],
        out_specs=(_node_spec(8), _node_spec(16)),
        out_shape=(jax.ShapeDtypeStruct((NPAD, 8), _F32),
                   jax.ShapeDtypeStruct((NPAD, 16), _F32)),
    )(p4, hs4, dinv, b4r, x1, x2, x3, c1a, c1b2, c1c, c1d)

    vals2d = jnp.concatenate(
        [x4[:N, 0], jnp.zeros((NB * 128 - N,), _F32)]).reshape(NB, 128)
    gidx = pl.pallas_call(
        _rank_body,
        out_shape=jax.ShapeDtypeStruct((SR, 128), _I32),
        scratch_shapes=[pltpu.VMEM((NB, 128), _I32)],
    )(vals2d, batchp)

    pool16 = _pool_gather(ypad, gidx)

    return pl.pallas_call(
        _classifier_body,
        out_shape=jax.ShapeDtypeStruct((B, 2), _F32),
    )(pool16, c1b.reshape(1, 16), c2wr, c2b, f1w, f1b, f2w, f2b)


# speedup vs baseline: 28.6350x; 1.0839x over previous
import functools

import jax
import jax.numpy as jnp
from jax import lax
from jax.experimental import pallas as pl
from jax.experimental.pallas import tpu as pltpu
from jax.experimental.pallas import tpu_sc as plsc

N = 10000
E = 320000
H = 32
B = 100
K = 100

NC = 2
NS = 16
NW = NC * NS
NPAD = N + 112
ECH = 128
NCH = 80
EPT = ECH * NCH
EP = EPT * NW
RPT = NPAD // NS
SR = 96
SLOTS = SR * 128
NB = 80
GB = 8
RB = NPAD // GB

_F32 = jnp.float32
_I32 = jnp.int32
_HI = jax.lax.Precision.HIGHEST


def _dot(a, b):
    return jnp.dot(a, b, preferred_element_type=_F32, precision=_HI)


def _sc_mesh():
    return plsc.VectorSubcoreMesh(core_axis_name="c", subcore_axis_name="s")


_D = 8


def _deg_pass(dstp, ones_rows, zrows):
    @functools.partial(
        pl.kernel,
        out_type=jax.ShapeDtypeStruct((NC, NPAD, 8), _F32),
        mesh=_sc_mesh(),
        compiler_params=pltpu.CompilerParams(use_tc_tiling_on_sc=False),
        scratch_types=[
            pltpu.VMEM((NCH, ECH), _I32),
            pltpu.VMEM((ECH, 8), _F32),
            pltpu.VMEM_SHARED((NPAD, 8), _F32),
        ] + [pltpu.SemaphoreType.DMA] * _D,
    )
    def k(dst_ref, ones_ref, z_ref, out_ref, didx, onesb, acc, *sems):
        cid = lax.axis_index("c")
        sid = lax.axis_index("s")
        base = (cid * NS + sid) * NCH
        pltpu.sync_copy(ones_ref, onesb)
        pltpu.sync_copy(z_ref, acc.at[pl.ds(sid * RPT, RPT)])
        pltpu.sync_copy(dst_ref.at[pl.ds(base, NCH)], didx)
        plsc.subcore_barrier()

        def scat(i, d):
            return pltpu.make_async_copy(onesb, acc.at[didx.at[i]], sems[d])

        def body(g, carry):
            i0 = g * _D
            for d in range(_D):
                scat(i0 + d, d).start(add=True)
            for d in range(_D):
                scat(i0 + d, d).wait()
            return carry

        lax.fori_loop(0, NCH // _D, body, 0)
        plsc.subcore_barrier()
        pltpu.sync_copy(acc.at[pl.ds(sid * RPT, RPT)],
                        out_ref.at[cid, pl.ds(sid * RPT, RPT)])

    return k(dstp, ones_rows, zrows)


def _edge_pass(hs, srcp, dstp, zrows, F):
    G = NCH // _D

    @functools.partial(
        pl.kernel,
        out_type=jax.ShapeDtypeStruct((NC, NPAD, F), _F32),
        mesh=_sc_mesh(),
        compiler_params=pltpu.CompilerParams(use_tc_tiling_on_sc=False),
        scratch_types=[
            pltpu.VMEM((NCH, ECH), _I32),
            pltpu.VMEM((NCH, ECH), _I32),
            pltpu.VMEM((_D, ECH, F), _F32),
            pltpu.VMEM_SHARED((NPAD, F), _F32),
        ] + [pltpu.SemaphoreType.DMA] * (2 * _D),
    )
    def k(hs_ref, src_ref, dst_ref, z_ref, out_ref, sidx, didx, rows, acc,
          *sems):
        gsem = sems[:_D]
        ssem = sems[_D:]
        cid = lax.axis_index("c")
        sid = lax.axis_index("s")
        base = (cid * NS + sid) * NCH
        pltpu.sync_copy(z_ref, acc.at[pl.ds(sid * RPT, RPT)])
        pltpu.sync_copy(src_ref.at[pl.ds(base, NCH)], sidx)
        pltpu.sync_copy(dst_ref.at[pl.ds(base, NCH)], didx)
        plsc.subcore_barrier()

        def gather(i, d):
            return pltpu.make_async_copy(hs_ref.at[sidx.at[i]], rows.at[d],
                                         gsem[d])

        def scat(i, d):
            return pltpu.make_async_copy(rows.at[d], acc.at[didx.at[i]],
                                         ssem[d])

        for d in range(_D):
            gather(d, d).start()

        def body(g, carry):
            i0 = g * _D
            for d in range(_D):
                gather(i0 + d, d).wait()
                scat(i0 + d, d).start(add=True)
            for d in range(_D):
                scat(i0 + d, d).wait()
                @pl.when(g < G - 1)
                def _():
                    gather(i0 + _D + d, d).start()
            return carry

        lax.fori_loop(0, G, body, 0)
        plsc.subcore_barrier()
        pltpu.sync_copy(acc.at[pl.ds(sid * RPT, RPT)],
                        out_ref.at[cid, pl.ds(sid * RPT, RPT)])

    return k(hs, srcp, dstp, zrows)


def _pool_gather(ypad, gidx):
    @functools.partial(
        pl.kernel,
        out_type=jax.ShapeDtypeStruct((SLOTS, 16), _F32),
        mesh=_sc_mesh(),
        compiler_params=pltpu.CompilerParams(use_tc_tiling_on_sc=False),
        scratch_types=[
            pltpu.VMEM((128,), _I32),
            pltpu.VMEM((128, 16), _F32),
        ],
    )
    def k(y_ref, g_ref, out_ref, idxb, rows):
        cid = lax.axis_index("c")
        sid = lax.axis_index("s")
        tid = cid * NS + sid
        for j in range(SR // NW):
            r = tid * (SR // NW) + j
            pltpu.sync_copy(g_ref.at[r], idxb)
            pltpu.sync_copy(y_ref.at[idxb], rows)
            pltpu.sync_copy(rows, out_ref.at[pl.ds(r * 128, 128)])

    return k(ypad, gidx)


def _row_mask(i, f):
    thresh = N - i * RB
    return lax.broadcasted_iota(_I32, (RB, f), 0) < thresh


def _t1_body(x_ref, w_ref, dp_ref, dinv_ref, hs_ref):
    i = pl.program_id(0)
    deg8 = dp_ref[0] + dp_ref[1] + 1.0
    dinv8 = lax.rsqrt(deg8)
    dinv32 = jnp.concatenate([dinv8] * 4, axis=1)
    dinv_ref[...] = dinv32
    h = _dot(x_ref[...], w_ref[...])
    hs_ref[...] = jnp.where(_row_mask(i, H), dinv32 * h, 0.0)


def _tmid_body(p_ref, hsp_ref, dinv_ref, b_ref, w_ref, x_ref, hs_ref):
    i = pl.program_id(0)
    fo = w_ref.shape[1]
    s = p_ref[0] + p_ref[1] + hsp_ref[...]
    xl = jnp.tanh(dinv_ref[...] * s + b_ref[...])
    x_ref[...] = xl
    h = _dot(xl, w_ref[...])
    hs = dinv_ref[...][:, :fo] * h
    hs_ref[...] = jnp.where(_row_mask(i, fo), hs, 0.0)


def _t5_body(p_ref, hs4_ref, dinv_ref, b4_ref, x1_ref, x2_ref, x3_ref,
             c1a_ref, c1b2_ref, c1c_ref, c1d_ref, x4_ref, y_ref):
    i = pl.program_id(0)
    s4 = p_ref[0] + p_ref[1] + hs4_ref[...]
    x4 = jnp.tanh(dinv_ref[...][:, 0:8] * s4 + b4_ref[...])
    x4_ref[...] = x4
    x4_16 = jnp.concatenate([x4, x4], axis=1)
    y = (_dot(x1_ref[...], c1a_ref[...]) +
         _dot(x2_ref[...], c1b2_ref[...]) +
         _dot(x3_ref[...], c1c_ref[...]) +
         x4_16 * c1d_ref[...])
    y_ref[...] = jnp.where(_row_mask(i, 16), y, 0.0)


def _node_spec(f):
    return pl.BlockSpec((RB, f), lambda i: (i, 0))


def _part_spec(f):
    return pl.BlockSpec((2, RB, f), lambda i: (0, i, 0))


def _full_spec(shape):
    nd = len(shape)
    return pl.BlockSpec(shape, lambda i: (0,) * nd)


def _rank_body(v_ref, b_ref, gidx_ref, slots_ref):
    iota128 = lax.broadcasted_iota(_I32, (1, 128), 1)
    eye = (lax.broadcasted_iota(_I32, (128, 128), 0) ==
           lax.broadcasted_iota(_I32, (128, 128), 1)).astype(_F32)

    def tcol(row_f32):
        return lax.dot_general(eye, row_f32, (((1,), (1,)), ((), ())),
                               preferred_element_type=_F32, precision=_HI)

    def trow(col_f32):
        return lax.dot_general(col_f32, eye, (((0,), (0,)), ((), ())),
                               preferred_element_type=_F32, precision=_HI)

    counts = jnp.zeros((1, 128), _F32)
    for r in range(NB):
        b_col = tcol(b_ref[r:r + 1, :].astype(_F32))
        counts = counts + jnp.sum(
            (b_col == iota128.astype(_F32)).astype(_F32),
            axis=0, keepdims=True)
    tri = (lax.broadcasted_iota(_I32, (128, 128), 0) <
           lax.broadcasted_iota(_I32, (128, 128), 1)).astype(_F32)
    starts = _dot(counts, tri)
    ends = starts + counts

    for ci in range(NB):
        vi = tcol(v_ref[ci:ci + 1, :])
        bi = tcol(b_ref[ci:ci + 1, :].astype(_F32))
        ii = ci * 128 + lax.broadcasted_iota(_I32, (128, 1), 0)
        g_first = b_ref[ci, 0]
        g_last = b_ref[ci, 127]
        gf = g_first.astype(_F32)
        gl = g_last.astype(_F32)
        jlo = jnp.sum(jnp.where(iota128.astype(_F32) == gf, starts, 0.0))
        jhi = jnp.sum(jnp.where(iota128.astype(_F32) == gl, ends, 0.0))
        klo = jnp.floor(jlo / 128.0).astype(_I32)
        khi = jnp.ceil(jhi / 128.0).astype(_I32)

        def jbody(kk, acc, vi=vi, bi=bi, ii=ii):
            vj = v_ref[pl.ds(kk, 1), :]
            bj = b_ref[pl.ds(kk, 1), :].astype(_F32)
            ij = kk * 128 + iota128
            gt = (vj > vi) | ((vj == vi) & (ij < ii))
            hit = (gt & (bj == bi)).astype(_F32)
            return acc + jnp.sum(hit, axis=1, keepdims=True)

        rank = lax.fori_loop(klo, khi, jbody,
                             jnp.zeros((128, 1), _F32))
        slot = jnp.where((bi < float(B)) & (rank < float(K)),
                         bi * float(K) + rank, -1.0)
        slots_ref[ci:ci + 1, :] = trow(slot).astype(_I32)

    for rc in range(SR):
        s0 = rc * 128
        g0 = min(s0 // K, B - 1)
        g1 = min((s0 + 127) // K, B - 1)
        jlo = starts[0, g0]
        jhi = ends[0, g1]
        klo = jnp.floor(jlo / 128.0).astype(_I32)
        khi = jnp.ceil(jhi / 128.0).astype(_I32)
        svec = s0 + lax.broadcasted_iota(_I32, (128, 1), 0)

        def jbody2(kk, carry, svec=svec):
            gi_acc, w_acc = carry
            sj = slots_ref[pl.ds(kk, 1), :]
            ij = (kk * 128 + iota128).astype(_F32)
            eqm = (sj == svec).astype(_F32)
            gi_acc = gi_acc + jnp.sum(eqm * ij, axis=1, keepdims=True)
            w_acc = w_acc + jnp.sum(eqm, axis=1, keepdims=True)
            return gi_acc, w_acc

        gi, w = lax.fori_loop(klo, khi, jbody2,
                              (jnp.zeros((128, 1), _F32),
                               jnp.zeros((128, 1), _F32)))
        fill = (N + (svec % 32)).astype(_F32)
        out = jnp.where(w > 0, gi, fill)
        gidx_ref[rc:rc + 1, :] = trow(out).astype(_I32)


def _classifier_body(pool_ref, c1b_ref, c2w_ref, c2b_ref,
                     f1w_ref, f1b_ref, f2w_ref, f2b_ref, out_ref):
    h1 = pool_ref[...][:B * K] + c1b_ref[...]
    h1 = jnp.maximum(h1, 0.0).reshape(B, K // 2, 2, 16)
    h2 = jnp.maximum(h1[:, :, 0, :], h1[:, :, 1, :])
    w5 = jnp.concatenate([h2[:, t:t + 46, :] for t in range(5)], axis=-1)
    h3 = lax.dot_general(w5, c2w_ref[...], (((2,), (0,)), ((), ())),
                         preferred_element_type=_F32, precision=_HI)
    h3 = jnp.maximum(h3 + c2b_ref[...][None, None, :], 0.0)
    h3 = jnp.transpose(h3, (0, 2, 1)).reshape(B, 32 * 46)
    h4 = jnp.maximum(_dot(h3, f1w_ref[...]) + f1b_ref[...][None, :], 0.0)
    lg = _dot(h4, f2w_ref[...]) + f2b_ref[...][None, :]
    m = jnp.max(lg, axis=-1, keepdims=True)
    lse = m + jnp.log(jnp.sum(jnp.exp(lg - m), axis=-1, keepdims=True))
    out_ref[...] = lg - lse


def kernel(x, edge_index, batch, W1, b1, W2, b2, W3, b3, W4, b4,
           c1w, c1b, c2w, c2b, f1w, f1b, f2w, f2b):
    src = edge_index[0]
    dst = edge_index[1]

    fill = (N + (jnp.arange(EP - E, dtype=_I32) % 32))
    srcp = jnp.concatenate([src, fill]).reshape(EP // ECH, ECH)
    dstp = jnp.concatenate([dst, fill]).reshape(EP // ECH, ECH)
    batchp = jnp.concatenate(
        [batch, jnp.full((NB * 128 - N,), B, _I32)]).reshape(NB, 128)
    xpad = jnp.concatenate([x, jnp.zeros((NPAD - N, x.shape[1]), _F32)])
    z32 = jnp.zeros((RPT, H), _F32)
    z8 = jnp.zeros((RPT, 8), _F32)
    ones8 = jnp.ones((ECH, 8), _F32)
    W4rep = jnp.repeat(W4, 8, axis=1)
    b1r, b2r, b3r = b1.reshape(1, H), b2.reshape(1, H), b3.reshape(1, H)
    b4r = jnp.repeat(b4.reshape(1, 1), 8, axis=1)
    c1wr = c1w.reshape(16, 3 * H + 1).T
    c1a, c1b2, c1c = c1wr[0:H], c1wr[H:2 * H], c1wr[2 * H:3 * H]
    c1d = c1wr[3 * H:3 * H + 1]
    c2wr = jnp.transpose(c2w, (2, 1, 0)).reshape(80, 32)

    dp = _deg_pass(dstp, ones8, z8)

    dinv, hs1 = pl.pallas_call(
        _t1_body,
        grid=(GB,),
        in_specs=[_node_spec(128), _full_spec((128, H)), _part_spec(8)],
        out_specs=(_node_spec(H), _node_spec(H)),
        out_shape=(jax.ShapeDtypeStruct((NPAD, H), _F32),
                   jax.ShapeDtypeStruct((NPAD, H), _F32)),
    )(xpad, W1, dp)
    p1 = _edge_pass(hs1, srcp, dstp, z32, H)

    def mid(p, hsp, b, w):
        fo = w.shape[1]
        return pl.pallas_call(
            _tmid_body,
            grid=(GB,),
            in_specs=[_part_spec(H), _node_spec(H), _node_spec(H),
                      _full_spec((1, H)), _full_spec((H, fo))],
            out_specs=(_node_spec(H), _node_spec(fo)),
            out_shape=(jax.ShapeDtypeStruct((NPAD, H), _F32),
                       jax.ShapeDtypeStruct((NPAD, fo), _F32)),
        )(p, hsp, dinv, b, w)

    x1, hs2 = mid(p1, hs1, b1r, W2)
    p2 = _edge_pass(hs2, srcp, dstp, z32, H)
    x2, hs3 = mid(p2, hs2, b2r, W3)
    p3 = _edge_pass(hs3, srcp, dstp, z32, H)
    x3, hs4 = mid(p3, hs3, b3r, W4rep)
    p4 = _edge_pass(hs4, srcp, dstp, z8, 8)

    x4, ypad = pl.pallas_call(
        _t5_body,
        grid=(GB,),
        in_specs=[_part_spec(8), _node_spec(8), _node_spec(H),
                  _full_spec((1, 8)), _node_spec(H), _node_spec(H),
                  _node_spec(H), _full_spec((H, 16)), _full_spec((H, 16)),
                  _full_spec((H, 16)), _full_spec((1, 16))],
        out_specs=(_node_spec(8), _node_spec(16)),
        out_shape=(jax.ShapeDtypeStruct((NPAD, 8), _F32),
                   jax.ShapeDtypeStruct((NPAD, 16), _F32)),
    )(p4, hs4, dinv, b4r, x1, x2, x3, c1a, c1b2, c1c, c1d)

    vals2d = jnp.concatenate(
        [x4[:N, 0], jnp.zeros((NB * 128 - N,), _F32)]).reshape(NB, 128)
    gidx = pl.pallas_call(
        _rank_body,
        out_shape=jax.ShapeDtypeStruct((SR, 128), _I32),
        scratch_shapes=[pltpu.VMEM((NB, 128), _I32)],
    )(vals2d, batchp)

    pool16 = _pool_gather(ypad, gidx)

    return pl.pallas_call(
        _classifier_body,
        out_shape=jax.ShapeDtypeStruct((B, 2), _F32),
    )(pool16, c1b.reshape(1, 16), c2wr, c2b, f1w, f1b, f2w, f2b)


# speedup vs baseline: 30.3881x; 1.0612x over previous
import functools

import jax
import jax.numpy as jnp
from jax import lax
from jax.experimental import pallas as pl
from jax.experimental.pallas import tpu as pltpu
from jax.experimental.pallas import tpu_sc as plsc

N = 10000
E = 320000
H = 32
B = 100
K = 100

NC = 2
NS = 16
NW = NC * NS
NPAD = N + 112
ECH = 128
NCH = 80
EPT = ECH * NCH
EP = EPT * NW
RPT = NPAD // NS
SR = 96
SLOTS = SR * 128
NB = 80
GB = 8
RB = NPAD // GB

_F32 = jnp.float32
_I32 = jnp.int32
_HI = jax.lax.Precision.HIGHEST


def _dot(a, b):
    return jnp.dot(a, b, preferred_element_type=_F32, precision=_HI)


def _sc_mesh():
    return plsc.VectorSubcoreMesh(core_axis_name="c", subcore_axis_name="s")


_D = 8


def _deg_pass(dstp, ones_rows, zrows):
    @functools.partial(
        pl.kernel,
        out_type=jax.ShapeDtypeStruct((NC, NPAD, 8), _F32),
        mesh=_sc_mesh(),
        compiler_params=pltpu.CompilerParams(use_tc_tiling_on_sc=False),
        scratch_types=[
            pltpu.VMEM((NCH, ECH), _I32),
            pltpu.VMEM((ECH, 8), _F32),
            pltpu.VMEM_SHARED((NPAD, 8), _F32),
        ] + [pltpu.SemaphoreType.DMA] * _D,
    )
    def k(dst_ref, ones_ref, z_ref, out_ref, didx, onesb, acc, *sems):
        cid = lax.axis_index("c")
        sid = lax.axis_index("s")
        base = (cid * NS + sid) * NCH
        pltpu.sync_copy(ones_ref, onesb)
        pltpu.sync_copy(z_ref, acc.at[pl.ds(sid * RPT, RPT)])
        pltpu.sync_copy(dst_ref.at[pl.ds(base, NCH)], didx)
        plsc.subcore_barrier()

        def scat(i, d):
            return pltpu.make_async_copy(onesb, acc.at[didx.at[i]], sems[d])

        def body(g, carry):
            i0 = g * _D
            for d in range(_D):
                scat(i0 + d, d).start(add=True)
            for d in range(_D):
                scat(i0 + d, d).wait()
            return carry

        lax.fori_loop(0, NCH // _D, body, 0)
        plsc.subcore_barrier()
        pltpu.sync_copy(acc.at[pl.ds(sid * RPT, RPT)],
                        out_ref.at[cid, pl.ds(sid * RPT, RPT)])

    return k(dstp, ones_rows, zrows)


def _edge_pass(hs, srcp, dstp, zrows, F):
    G = NCH // _D

    @functools.partial(
        pl.kernel,
        out_type=jax.ShapeDtypeStruct((NC, NPAD, F), _F32),
        mesh=_sc_mesh(),
        compiler_params=pltpu.CompilerParams(use_tc_tiling_on_sc=False),
        scratch_types=[
            pltpu.VMEM((NCH, ECH), _I32),
            pltpu.VMEM((NCH, ECH), _I32),
            pltpu.VMEM((_D, ECH, F), _F32),
            pltpu.VMEM_SHARED((NPAD, F), _F32),
        ] + [pltpu.SemaphoreType.DMA] * (2 * _D),
    )
    def k(hs_ref, src_ref, dst_ref, z_ref, out_ref, sidx, didx, rows, acc,
          *sems):
        gsem = sems[:_D]
        ssem = sems[_D:]
        cid = lax.axis_index("c")
        sid = lax.axis_index("s")
        base = (cid * NS + sid) * NCH
        pltpu.sync_copy(z_ref, acc.at[pl.ds(sid * RPT, RPT)])
        pltpu.sync_copy(src_ref.at[pl.ds(base, NCH)], sidx)
        pltpu.sync_copy(dst_ref.at[pl.ds(base, NCH)], didx)
        plsc.subcore_barrier()

        def gather(i, d):
            return pltpu.make_async_copy(hs_ref.at[sidx.at[i]], rows.at[d],
                                         gsem[d])

        def scat(i, d):
            return pltpu.make_async_copy(rows.at[d], acc.at[didx.at[i]],
                                         ssem[d])

        for d in range(_D):
            gather(d, d).start()

        def body(g, carry):
            i0 = g * _D
            for d in range(_D):
                gather(i0 + d, d).wait()
                scat(i0 + d, d).start(add=True)
            for d in range(_D):
                scat(i0 + d, d).wait()
                @pl.when(g < G - 1)
                def _():
                    gather(i0 + _D + d, d).start()
            return carry

        lax.fori_loop(0, G, body, 0)
        plsc.subcore_barrier()
        pltpu.sync_copy(acc.at[pl.ds(sid * RPT, RPT)],
                        out_ref.at[cid, pl.ds(sid * RPT, RPT)])

    return k(hs, srcp, dstp, zrows)


def _pool_gather(ypad, gidx):
    @functools.partial(
        pl.kernel,
        out_type=jax.ShapeDtypeStruct((SLOTS, 16), _F32),
        mesh=_sc_mesh(),
        compiler_params=pltpu.CompilerParams(use_tc_tiling_on_sc=False),
        scratch_types=[
            pltpu.VMEM((128,), _I32),
            pltpu.VMEM((128, 16), _F32),
        ],
    )
    def k(y_ref, g_ref, out_ref, idxb, rows):
        cid = lax.axis_index("c")
        sid = lax.axis_index("s")
        tid = cid * NS + sid
        for j in range(SR // NW):
            r = tid * (SR // NW) + j
            pltpu.sync_copy(g_ref.at[r], idxb)
            pltpu.sync_copy(y_ref.at[idxb], rows)
            pltpu.sync_copy(rows, out_ref.at[pl.ds(r * 128, 128)])

    return k(ypad, gidx)


def _row_mask(i, f):
    thresh = N - i * RB
    return lax.broadcasted_iota(_I32, (RB, f), 0) < thresh


def _t1_body(x_ref, w_ref, dp_ref, dinv_ref, hs_ref):
    i = pl.program_id(0)
    deg8 = dp_ref[0] + dp_ref[1] + 1.0
    dinv8 = lax.rsqrt(deg8)
    dinv32 = jnp.concatenate([dinv8] * 4, axis=1)
    dinv_ref[...] = dinv32
    h = _dot(x_ref[...], w_ref[...])
    hs_ref[...] = jnp.where(_row_mask(i, H), dinv32 * h, 0.0)


def _tmid_body(p_ref, hsp_ref, dinv_ref, b_ref, w_ref, x_ref, hs_ref):
    i = pl.program_id(0)
    fo = w_ref.shape[1]
    s = p_ref[0] + p_ref[1] + hsp_ref[...]
    xl = jnp.tanh(dinv_ref[...] * s + b_ref[...])
    x_ref[...] = xl
    h = _dot(xl, w_ref[...])
    hs = dinv_ref[...][:, :fo] * h
    hs_ref[...] = jnp.where(_row_mask(i, fo), hs, 0.0)


def _t5_body(p_ref, hs4_ref, dinv_ref, b4_ref, x1_ref, x2_ref, x3_ref,
             c1a_ref, c1b2_ref, c1c_ref, c1d_ref, x4_ref, y_ref):
    i = pl.program_id(0)
    s4 = p_ref[0] + p_ref[1] + hs4_ref[...]
    x4 = jnp.tanh(dinv_ref[...][:, 0:8] * s4 + b4_ref[...])
    x4_ref[...] = x4
    x4_16 = jnp.concatenate([x4, x4], axis=1)
    y = (_dot(x1_ref[...], c1a_ref[...]) +
         _dot(x2_ref[...], c1b2_ref[...]) +
         _dot(x3_ref[...], c1c_ref[...]) +
         x4_16 * c1d_ref[...])
    y_ref[...] = jnp.where(_row_mask(i, 16), y, 0.0)


def _node_spec(f):
    return pl.BlockSpec((RB, f), lambda i: (i, 0))


def _part_spec(f):
    return pl.BlockSpec((2, RB, f), lambda i: (0, i, 0))


def _full_spec(shape):
    nd = len(shape)
    return pl.BlockSpec(shape, lambda i: (0,) * nd)


def _rank_body(v_ref, b_ref, gidx_ref, slots_ref):
    iota128 = lax.broadcasted_iota(_I32, (1, 128), 1)
    eye = (lax.broadcasted_iota(_I32, (128, 128), 0) ==
           lax.broadcasted_iota(_I32, (128, 128), 1)).astype(_F32)

    def tcol(row_f32):
        return lax.dot_general(eye, row_f32, (((1,), (1,)), ((), ())),
                               preferred_element_type=_F32, precision=_HI)

    def trow(col_f32):
        return lax.dot_general(col_f32, eye, (((0,), (0,)), ((), ())),
                               preferred_element_type=_F32, precision=_HI)

    iota_col = lax.broadcasted_iota(_I32, (128, 1), 0)
    counts_c = jnp.zeros((128, 1), _F32)
    for r in range(NB):
        counts_c = counts_c + jnp.sum(
            (b_ref[r:r + 1, :] == iota_col).astype(_F32),
            axis=1, keepdims=True)
    tri = (lax.broadcasted_iota(_I32, (128, 128), 1) <
           lax.broadcasted_iota(_I32, (128, 128), 0)).astype(_F32)
    starts_c = _dot(tri, counts_c)
    ends_c = starts_c + counts_c
    starts = trow(starts_c)
    ends = trow(ends_c)

    for ci in range(NB):
        vi = tcol(v_ref[ci:ci + 1, :])
        bi = tcol(b_ref[ci:ci + 1, :].astype(_F32))
        ii = ci * 128 + lax.broadcasted_iota(_I32, (128, 1), 0)
        g_first = b_ref[ci, 0]
        g_last = b_ref[ci, 127]
        gf = g_first.astype(_F32)
        gl = g_last.astype(_F32)
        jlo = jnp.sum(jnp.where(iota128.astype(_F32) == gf, starts, 0.0))
        jhi = jnp.sum(jnp.where(iota128.astype(_F32) == gl, ends, 0.0))
        klo = jnp.floor(jlo / 128.0).astype(_I32)
        khi = jnp.ceil(jhi / 128.0).astype(_I32)

        def jbody(kk, acc, vi=vi, bi=bi, ii=ii):
            vj = v_ref[pl.ds(kk, 1), :]
            bj = b_ref[pl.ds(kk, 1), :].astype(_F32)
            ij = kk * 128 + iota128
            gt = (vj > vi) | ((vj == vi) & (ij < ii))
            hit = (gt & (bj == bi)).astype(_F32)
            return acc + hit

        rank = jnp.sum(
            lax.fori_loop(klo, khi, jbody, jnp.zeros((128, 128), _F32)),
            axis=1, keepdims=True)
        slot = jnp.where((bi < float(B)) & (rank < float(K)),
                         bi * float(K) + rank, -1.0)
        slots_ref[ci:ci + 1, :] = trow(slot).astype(_I32)

    for rc in range(SR):
        s0 = rc * 128
        g0 = min(s0 // K, B - 1)
        g1 = min((s0 + 127) // K, B - 1)
        jlo = starts[0, g0]
        jhi = ends[0, g1]
        klo = jnp.floor(jlo / 128.0).astype(_I32)
        khi = jnp.ceil(jhi / 128.0).astype(_I32)
        svec = s0 + lax.broadcasted_iota(_I32, (128, 1), 0)

        def jbody2(kk, carry, svec=svec):
            gi_acc, w_acc = carry
            sj = slots_ref[pl.ds(kk, 1), :]
            ij = (kk * 128 + iota128).astype(_F32)
            eqm = (sj == svec).astype(_F32)
            return gi_acc + eqm * ij, w_acc + eqm

        gi2, w2 = lax.fori_loop(klo, khi, jbody2,
                                (jnp.zeros((128, 128), _F32),
                                 jnp.zeros((128, 128), _F32)))
        gi = jnp.sum(gi2, axis=1, keepdims=True)
        w = jnp.sum(w2, axis=1, keepdims=True)
        fill = (N + (svec % 32)).astype(_F32)
        out = jnp.where(w > 0, gi, fill)
        gidx_ref[rc:rc + 1, :] = trow(out).astype(_I32)


def _classifier_body(pool_ref, c1b_ref, c2w_ref, c2b_ref,
                     f1w_ref, f1b_ref, f2w_ref, f2b_ref, out_ref):
    h1 = pool_ref[...][:B * K] + c1b_ref[...]
    h1 = jnp.maximum(h1, 0.0).reshape(B, K // 2, 2, 16)
    h2 = jnp.maximum(h1[:, :, 0, :], h1[:, :, 1, :])
    w5 = jnp.concatenate([h2[:, t:t + 46, :] for t in range(5)], axis=-1)
    h3 = lax.dot_general(w5, c2w_ref[...], (((2,), (0,)), ((), ())),
                         preferred_element_type=_F32, precision=_HI)
    h3 = jnp.maximum(h3 + c2b_ref[...][None, None, :], 0.0)
    h3 = jnp.transpose(h3, (0, 2, 1)).reshape(B, 32 * 46)
    h4 = jnp.maximum(_dot(h3, f1w_ref[...]) + f1b_ref[...][None, :], 0.0)
    lg = _dot(h4, f2w_ref[...]) + f2b_ref[...][None, :]
    m = jnp.max(lg, axis=-1, keepdims=True)
    lse = m + jnp.log(jnp.sum(jnp.exp(lg - m), axis=-1, keepdims=True))
    out_ref[...] = lg - lse


def kernel(x, edge_index, batch, W1, b1, W2, b2, W3, b3, W4, b4,
           c1w, c1b, c2w, c2b, f1w, f1b, f2w, f2b):
    src = edge_index[0]
    dst = edge_index[1]

    fill = (N + (jnp.arange(EP - E, dtype=_I32) % 32))
    srcp = jnp.concatenate([src, fill]).reshape(EP // ECH, ECH)
    dstp = jnp.concatenate([dst, fill]).reshape(EP // ECH, ECH)
    batchp = jnp.concatenate(
        [batch, jnp.full((NB * 128 - N,), B, _I32)]).reshape(NB, 128)
    xpad = jnp.concatenate([x, jnp.zeros((NPAD - N, x.shape[1]), _F32)])
    z32 = jnp.zeros((RPT, H), _F32)
    z8 = jnp.zeros((RPT, 8), _F32)
    ones8 = jnp.ones((ECH, 8), _F32)
    W4rep = jnp.repeat(W4, 8, axis=1)
    b1r, b2r, b3r = b1.reshape(1, H), b2.reshape(1, H), b3.reshape(1, H)
    b4r = jnp.repeat(b4.reshape(1, 1), 8, axis=1)
    c1wr = c1w.reshape(16, 3 * H + 1).T
    c1a, c1b2, c1c = c1wr[0:H], c1wr[H:2 * H], c1wr[2 * H:3 * H]
    c1d = c1wr[3 * H:3 * H + 1]
    c2wr = jnp.transpose(c2w, (2, 1, 0)).reshape(80, 32)

    dp = _deg_pass(dstp, ones8, z8)

    dinv, hs1 = pl.pallas_call(
        _t1_body,
        grid=(GB,),
        in_specs=[_node_spec(128), _full_spec((128, H)), _part_spec(8)],
        out_specs=(_node_spec(H), _node_spec(H)),
        out_shape=(jax.ShapeDtypeStruct((NPAD, H), _F32),
                   jax.ShapeDtypeStruct((NPAD, H), _F32)),
    )(xpad, W1, dp)
    p1 = _edge_pass(hs1, srcp, dstp, z32, H)

    def mid(p, hsp, b, w):
        fo = w.shape[1]
        return pl.pallas_call(
            _tmid_body,
            grid=(GB,),
            in_specs=[_part_spec(H), _node_spec(H), _node_spec(H),
                      _full_spec((1, H)), _full_spec((H, fo))],
            out_specs=(_node_spec(H), _node_spec(fo)),
            out_shape=(jax.ShapeDtypeStruct((NPAD, H), _F32),
                       jax.ShapeDtypeStruct((NPAD, fo), _F32)),
        )(p, hsp, dinv, b, w)

    x1, hs2 = mid(p1, hs1, b1r, W2)
    p2 = _edge_pass(hs2, srcp, dstp, z32, H)
    x2, hs3 = mid(p2, hs2, b2r, W3)
    p3 = _edge_pass(hs3, srcp, dstp, z32, H)
    x3, hs4 = mid(p3, hs3, b3r, W4rep)
    p4 = _edge_pass(hs4, srcp, dstp, z8, 8)

    x4, ypad = pl.pallas_call(
        _t5_body,
        grid=(GB,),
        in_specs=[_part_spec(8), _node_spec(8), _node_spec(H),
                  _full_spec((1, 8)), _node_spec(H), _node_spec(H),
                  _node_spec(H), _full_spec((H, 16)), _full_spec((H, 16)),
                  _full_spec((H, 16)), _full_spec((1, 16))],
        out_specs=(_node_spec(8), _node_spec(16)),
        out_shape=(jax.ShapeDtypeStruct((NPAD, 8), _F32),
                   jax.ShapeDtypeStruct((NPAD, 16), _F32)),
    )(p4, hs4, dinv, b4r, x1, x2, x3, c1a, c1b2, c1c, c1d)

    vals2d = jnp.concatenate(
        [x4[:N, 0], jnp.zeros((NB * 128 - N,), _F32)]).reshape(NB, 128)
    gidx = pl.pallas_call(
        _rank_body,
        out_shape=jax.ShapeDtypeStruct((SR, 128), _I32),
        scratch_shapes=[pltpu.VMEM((NB, 128), _I32)],
    )(vals2d, batchp)

    pool16 = _pool_gather(ypad, gidx)

    return pl.pallas_call(
        _classifier_body,
        out_shape=jax.ShapeDtypeStruct((B, 2), _F32),
    )(pool16, c1b.reshape(1, 16), c2wr, c2b, f1w, f1b, f2w, f2b)
